# baseline (speedup 1.0000x reference)
import functools

import jax
import jax.numpy as jnp
from jax import lax
from jax.experimental import pallas as pl
from jax.experimental.pallas import tpu as pltpu
from jax.experimental.pallas import tpu_sc as plsc

D = 128
ROW = D + 1
ROWP = 144

_GATHER_DNUMS = lax.GatherDimensionNumbers(
    offset_dims=(), collapsed_slice_dims=(0,), start_index_map=(0,))


def _shuffle(x, idx):
    return lax.gather(x, idx[:, None], _GATHER_DNUMS, (1,),
                      mode=lax.GatherScatterMode.PROMISE_IN_BOUNDS)


def _bf16_round(x):
    t = x * 65537.0
    return t - (t - x)


def _allsum(x):
    lane = lax.iota(jnp.int32, 16)
    for sh in (8, 4, 2, 1):
        x = x + _shuffle(x, lane ^ sh)
    return x


def _make_sc_kernel(R, rows_per_worker, chunk):
    n_chunks = rows_per_worker // chunk
    mesh = plsc.VectorSubcoreMesh(core_axis_name="c", subcore_axis_name="s")
    info = plsc.get_sparse_core_info()
    num_cores = info.num_cores

    @functools.partial(
        pl.kernel,
        mesh=mesh,
        out_type=[
            jax.ShapeDtypeStruct((R, ROWP), jnp.float32),
            jax.ShapeDtypeStruct((R, ROWP), jnp.float32),
            jax.ShapeDtypeStruct((R,), jnp.float32),
            jax.ShapeDtypeStruct((R,), jnp.float32),
        ],
        scratch_types=[
            pltpu.VMEM((chunk, ROWP), jnp.float32),
            pltpu.VMEM((chunk, ROWP), jnp.float32),
            pltpu.VMEM((D,), jnp.float32),
            pltpu.VMEM((D,), jnp.float32),
            pltpu.VMEM((rows_per_worker,), jnp.float32),
            pltpu.VMEM((rows_per_worker,), jnp.float32),
        ],
    )
    def sc_kernel(l_hbm, u_hbm, lb_hbm, ub_hbm,
                  pl_hbm, pu_hbm, clb_hbm, cub_hbm,
                  l_v, u_v, lb_v, ub_v, clb_v, cub_v):
        wid = lax.axis_index("s") * num_cores + lax.axis_index("c")
        row0 = wid * rows_per_worker
        pltpu.sync_copy(lb_hbm, lb_v)
        pltpu.sync_copy(ub_hbm, ub_v)
        for dd in range(D // 16):
            sl = pl.ds(dd * 16, 16)
            lb_v[sl] = _bf16_round(lb_v[sl])
            ub_v[sl] = _bf16_round(ub_v[sl])
        lane = lax.iota(jnp.int32, 16)
        is_bias = lane == 0

        def chunk_body(ci, _):
            base = row0 + ci * chunk
            pltpu.sync_copy(l_hbm.at[pl.ds(base, chunk)], l_v)
            pltpu.sync_copy(u_hbm.at[pl.ds(base, chunk)], u_v)

            def row_compute(r):
                tail_l = l_v[r, pl.ds(D, 16)]
                tail_u = u_v[r, pl.ds(D, 16)]
                bias_l = jnp.where(is_bias, tail_l, 0.0)
                bias_u = jnp.where(is_bias, tail_u, 0.0)
                amin_l = bias_l
                amax_l = bias_l
                amin_u = bias_u
                amax_u = bias_u
                for dd in range(D // 16):
                    lo = lb_v[pl.ds(dd * 16, 16)]
                    hi = ub_v[pl.ds(dd * 16, 16)]
                    wl = _bf16_round(l_v[r, pl.ds(dd * 16, 16)])
                    p1 = wl * lo
                    p2 = wl * hi
                    amin_l = amin_l + jnp.minimum(p1, p2)
                    amax_l = amax_l + jnp.maximum(p1, p2)
                    wu = _bf16_round(u_v[r, pl.ds(dd * 16, 16)])
                    q1 = wu * lo
                    q2 = wu * hi
                    amin_u = amin_u + jnp.minimum(q1, q2)
                    amax_u = amax_u + jnp.maximum(q1, q2)
                conc_lb = _allsum(amin_l)
                max_lb = _allsum(amax_l)
                min_ub = _allsum(amin_u)
                conc_ub = _allsum(amax_u)

                inactive = conc_ub <= 0.0
                unstable = (conc_lb < 0.0) & (conc_ub > 0.0)
                m_inact = unstable & (
                    (jnp.abs(conc_lb) > jnp.abs(conc_ub)) | (max_lb <= 0.0))
                m_act = unstable & (jnp.abs(conc_lb) <= jnp.abs(conc_ub))
                den_l = jnp.where(m_act, max_lb - conc_lb, 1.0)
                den_l = jnp.where(den_l == 0.0, 1.0, den_l)
                a_l = jnp.where(max_lb < 0.0, 0.0, max_lb / den_l)
                s_l = jnp.where(m_act, a_l,
                                jnp.where(inactive | m_inact, 0.0, 1.0))

                zc = unstable & (min_ub <= 0.0)
                den_u = jnp.where(zc, conc_ub - min_ub, 1.0)
                den_u = jnp.where(den_u == 0.0, 1.0, den_u)
                a_u = conc_ub / den_u
                s_u = jnp.where(zc, a_u, jnp.where(inactive, 0.0, 1.0))
                b_adj = jnp.where(zc, -a_u * min_ub, 0.0)

                for dd in range(D // 16):
                    sl = pl.ds(dd * 16, 16)
                    l_v[r, sl] = s_l * l_v[r, sl]
                    u_v[r, sl] = s_u * u_v[r, sl]
                l_v[r, pl.ds(D, 16)] = s_l * tail_l
                u_v[r, pl.ds(D, 16)] = (
                    s_u * tail_u + jnp.where(is_bias, b_adj, 0.0))
                return (jnp.maximum(s_l * conc_lb, 0.0),
                        jnp.maximum(s_u * conc_ub + b_adj, 0.0))

            def group_body(g, _):
                gbase = g * 16
                clb_acc = jnp.zeros((16,), jnp.float32)
                cub_acc = jnp.zeros((16,), jnp.float32)
                for j in range(16):
                    clb_j, cub_j = row_compute(gbase + j)
                    sel = lane == j
                    clb_acc = jnp.where(sel, clb_j, clb_acc)
                    cub_acc = jnp.where(sel, cub_j, cub_acc)
                obase = ci * chunk + gbase
                clb_v[pl.ds(obase, 16)] = clb_acc
                cub_v[pl.ds(obase, 16)] = cub_acc
                return 0

            lax.fori_loop(0, chunk // 16, group_body, 0)
            pltpu.sync_copy(l_v, pl_hbm.at[pl.ds(base, chunk)])
            pltpu.sync_copy(u_v, pu_hbm.at[pl.ds(base, chunk)])
            return 0

        lax.fori_loop(0, n_chunks, chunk_body, 0)
        pltpu.sync_copy(clb_v, clb_hbm.at[pl.ds(row0, rows_per_worker)])
        pltpu.sync_copy(cub_v, cub_hbm.at[pl.ds(row0, rows_per_worker)])

    return sc_kernel


def kernel(l, u, input_lb, input_ub):
    B, N, row = l.shape
    R = B * N
    n_workers = 32
    rows_per_worker = R // n_workers
    lp = jnp.pad(l.reshape(R, row), ((0, 0), (0, ROWP - row)))
    up = jnp.pad(u.reshape(R, row), ((0, 0), (0, ROWP - row)))
    sc = _make_sc_kernel(R, rows_per_worker, chunk=64)
    post_l, post_u, clb, cub = sc(lp, up, input_lb, input_ub)
    return (post_l[:, :row].reshape(B, N, row),
            post_u[:, :row].reshape(B, N, row),
            clb.reshape(B, N), cub.reshape(B, N))
